# baseline (speedup 1.0000x reference)
import jax
import jax.experimental.layout
import jax.numpy as jnp
from jax import lax
from jax.experimental import pallas as pl
from jax.experimental.pallas import tpu as pltpu
from jax.experimental.pallas import tpu_sc as plsc

W_ADJ = 0.4
R_ADJ = 10.0

NC = 2
NS = 16
LANES = 16
NW = NC * NS

VP = 100352
LP = 200
PHASE = 13
B = 16384
ROWS_PER_W = B // NW
GROUPS_PER_CHUNK = 2
CHUNK_ROWS = GROUPS_PER_CHUNK * LANES
CHUNKS = ROWS_PER_W // CHUNK_ROWS
UNROLL = 10


def _prep_body(w_ref, r0_ref, r1_ref, s_ref):
    s_ref[...] = (w_ref[...] + W_ADJ) * (r1_ref[...] - r0_ref[...]) / R_ADJ


def _build_s_table(w, r0, r1):
    shaped = jax.ShapeDtypeStruct((VP // 128, 128), jnp.float32)
    f = pl.pallas_call(_prep_body, out_shape=shaped)
    return f(
        w.reshape(VP // 128, 128),
        r0.reshape(VP // 128, 128),
        r1.reshape(VP // 128, 128),
    ).reshape(VP)


def _sc_body(s_hbm, idx_hbm, out_hbm, table_v, idx_v0, idx_v1, out_v,
             sem0, sem1):
    c = lax.axis_index("c")
    s = lax.axis_index("s")
    wid = s * NC + c
    row_base = wid * ROWS_PER_W

    pltpu.sync_copy(s_hbm, table_v)

    iota = lax.iota(jnp.int32, LANES)
    thresh = LP - PHASE * iota

    phase = PHASE * iota

    bufs = (idx_v0, idx_v1)
    sems = (sem0, sem1)

    def fire(chunk):
        row0 = row_base + chunk * CHUNK_ROWS
        return pltpu.async_copy(
            idx_hbm.at[pl.ds(row0, CHUNK_ROWS)], bufs[chunk % 2],
            sems[chunk % 2])

    desc = fire(0)
    for chunk in range(CHUNKS):
        buf = bufs[chunk % 2]
        row0 = row_base + chunk * CHUNK_ROWS
        desc.wait()
        if chunk + 1 < CHUNKS:
            desc = fire(chunk + 1)
        for g in range(GROUPS_PER_CHUNK):
            rows = iota + g * LANES

            def body(j, acc, buf=buf, rows=rows):
                vs = []
                for k in range(UNROLL):
                    jj = j + k
                    pos = phase + jj
                    col = jnp.where(thresh <= jj, pos - LP, pos)
                    iv = plsc.load_gather(buf, [rows, col])
                    vs.append(plsc.load_gather(table_v, [iv]))
                while len(vs) > 1:
                    rest = [vs[-1]] if len(vs) % 2 else []
                    vs = [a + b for a, b in zip(vs[::2], vs[1::2])] + rest
                return acc + vs[0]

            d = plsc.parallel_loop(
                0, LP, UNROLL, carry=jnp.zeros((LANES,), jnp.float32))(body)

            e = jnp.exp(-jnp.abs(d))
            inv = 1.0 / (1.0 + e)
            phi = inv
            plo = e * inv
            pos_m = d >= 0.0
            out0 = jnp.where(pos_m, plo, phi)
            out1 = jnp.where(pos_m, phi, plo)

            rows_l = iota + g * LANES
            plsc.store_scatter(out_v, [rows_l, jnp.zeros((LANES,), jnp.int32)],
                               out0)
            plsc.store_scatter(out_v, [rows_l, jnp.ones((LANES,), jnp.int32)],
                               out1)

        pltpu.sync_copy(out_v, out_hbm.at[pl.ds(row0, CHUNK_ROWS)])


def _kernel_impl(feat_idx, w_table, r_table):
    nb, nl = feat_idx.shape
    v1 = w_table.shape[0]
    pad_v = VP - v1

    w = jnp.pad(w_table[:, 0], (0, pad_v))
    r0 = jnp.pad(r_table[:, 0], (0, pad_v))
    r1 = jnp.pad(r_table[:, 1], (0, pad_v))
    s_table = _build_s_table(w, r0, r1)

    idx2d = feat_idx.astype(jnp.int32)

    mesh = plsc.VectorSubcoreMesh(core_axis_name="c", subcore_axis_name="s")
    sc = pl.kernel(
        _sc_body,
        out_type=jax.ShapeDtypeStruct((nb, 2), jnp.float32),
        mesh=mesh,
        scratch_types=[
            pltpu.VMEM((VP,), jnp.float32),
            pltpu.VMEM((CHUNK_ROWS, LP), jnp.int32),
            pltpu.VMEM((CHUNK_ROWS, LP), jnp.int32),
            pltpu.VMEM((CHUNK_ROWS, 2), jnp.float32),
            pltpu.SemaphoreType.DMA,
            pltpu.SemaphoreType.DMA,
        ],
        compiler_params=pltpu.CompilerParams(needs_layout_passes=False),
    )
    return sc(s_table, idx2d)


_jitted = None


def kernel(feat_idx, w_table, r_table):
    global _jitted
    if _jitted is None:
        try:
            dev = next(iter(feat_idx.devices()))
        except Exception:
            dev = jax.devices()[0]
        fmt = jax.experimental.layout.Format(
            jax.experimental.layout.Layout((1, 0), ((8, 128),)),
            jax.sharding.SingleDeviceSharding(dev))
        _jitted = jax.jit(_kernel_impl, out_shardings=fmt)
    return _jitted(feat_idx, w_table, r_table)
